# baseline (speedup 1.0000x reference)
import jax
import jax.numpy as jnp
from jax.experimental import pallas as pl
from jax.experimental.pallas import tpu as pltpu

NUM_EXPERTS = 64
TOP_K = 8
BLOCK_T = 256


def _router_block(x_ref, wt_ref, logits_ref, scores_ref, ew_ref, ei_ref):
    x = x_ref[...]
    wt = wt_ref[...]
    logits = jnp.dot(x, wt, preferred_element_type=jnp.float32)
    m = jnp.max(logits, axis=-1, keepdims=True)
    e = jnp.exp(logits - m)
    scores = e / jnp.sum(e, axis=-1, keepdims=True)
    logits_ref[...] = logits
    scores_ref[...] = scores

    s = scores
    colf = jax.lax.broadcasted_iota(jnp.int32, s.shape, 1).astype(jnp.float32)
    big = jnp.float32(NUM_EXPERTS)
    vals = []
    idxs = []
    for _ in range(TOP_K):
        mk = jnp.max(s, axis=-1, keepdims=True)
        ik = jnp.min(jnp.where(s == mk, colf, big), axis=-1, keepdims=True)
        vals.append(mk)
        idxs.append(ik)
        s = jnp.where(colf == ik, jnp.float32(-1.0), s)
    tv = jnp.concatenate(vals, axis=-1)
    ti = jnp.concatenate(idxs, axis=-1)
    ee = jnp.exp(tv - tv[:, :1])
    ew_ref[...] = ee / jnp.sum(ee, axis=-1, keepdims=True)
    ei_ref[...] = ti.astype(jnp.int32)


def kernel(x, W):
    bs, sq, d = x.shape
    n_tok = bs * sq
    x2 = x.reshape(n_tok, d)
    wt = W.T
    grid = (n_tok // BLOCK_T,)
    logits, scores, ew, ei = pl.pallas_call(
        _router_block,
        grid=grid,
        in_specs=[
            pl.BlockSpec((BLOCK_T, d), lambda i: (i, 0)),
            pl.BlockSpec((d, NUM_EXPERTS), lambda i: (0, 0)),
        ],
        out_specs=(
            pl.BlockSpec((BLOCK_T, NUM_EXPERTS), lambda i: (i, 0)),
            pl.BlockSpec((BLOCK_T, NUM_EXPERTS), lambda i: (i, 0)),
            pl.BlockSpec((BLOCK_T, TOP_K), lambda i: (i, 0)),
            pl.BlockSpec((BLOCK_T, TOP_K), lambda i: (i, 0)),
        ),
        out_shape=(
            jax.ShapeDtypeStruct((n_tok, NUM_EXPERTS), jnp.float32),
            jax.ShapeDtypeStruct((n_tok, NUM_EXPERTS), jnp.float32),
            jax.ShapeDtypeStruct((n_tok, TOP_K), jnp.float32),
            jax.ShapeDtypeStruct((n_tok, TOP_K), jnp.int32),
        ),
    )(x2, wt)
    return scores, logits, ew, ei


# speedup vs baseline: 1.3970x; 1.3970x over previous
import jax
import jax.numpy as jnp
from jax.experimental import pallas as pl
from jax.experimental.pallas import tpu as pltpu

NUM_EXPERTS = 64
TOP_K = 8
BLOCK_T = 1024


def _router_block(x_ref, wt_ref, logits_ref, scores_ref, ew_ref, ei_ref):
    x = x_ref[...]
    wt = wt_ref[...]
    logits = jnp.dot(x, wt, preferred_element_type=jnp.float32)
    m = jnp.max(logits, axis=-1, keepdims=True)
    e = jnp.exp(logits - m)
    scores = e / jnp.sum(e, axis=-1, keepdims=True)
    logits_ref[...] = logits
    scores_ref[...] = scores

    s = scores
    colf = jax.lax.broadcasted_iota(jnp.int32, s.shape, 1).astype(jnp.float32)
    big = jnp.float32(NUM_EXPERTS)
    vals = []
    idxs = []
    for _ in range(TOP_K):
        mk = jnp.max(s, axis=-1, keepdims=True)
        ik = jnp.min(jnp.where(s == mk, colf, big), axis=-1, keepdims=True)
        vals.append(mk)
        idxs.append(ik)
        s = jnp.where(colf == ik, jnp.float32(-1.0), s)
    tv = jnp.concatenate(vals, axis=-1)
    ti = jnp.concatenate(idxs, axis=-1)
    ee = jnp.exp(tv - tv[:, :1])
    ew_ref[...] = ee / jnp.sum(ee, axis=-1, keepdims=True)
    ei_ref[...] = ti.astype(jnp.int32)


def kernel(x, W):
    bs, sq, d = x.shape
    n_tok = bs * sq
    x2 = x.reshape(n_tok, d)
    wt = W.T
    grid = (n_tok // BLOCK_T,)
    logits, scores, ew, ei = pl.pallas_call(
        _router_block,
        grid=grid,
        in_specs=[
            pl.BlockSpec((BLOCK_T, d), lambda i: (i, 0)),
            pl.BlockSpec((d, NUM_EXPERTS), lambda i: (0, 0)),
        ],
        out_specs=(
            pl.BlockSpec((BLOCK_T, NUM_EXPERTS), lambda i: (i, 0)),
            pl.BlockSpec((BLOCK_T, NUM_EXPERTS), lambda i: (i, 0)),
            pl.BlockSpec((BLOCK_T, TOP_K), lambda i: (i, 0)),
            pl.BlockSpec((BLOCK_T, TOP_K), lambda i: (i, 0)),
        ),
        out_shape=(
            jax.ShapeDtypeStruct((n_tok, NUM_EXPERTS), jnp.float32),
            jax.ShapeDtypeStruct((n_tok, NUM_EXPERTS), jnp.float32),
            jax.ShapeDtypeStruct((n_tok, TOP_K), jnp.float32),
            jax.ShapeDtypeStruct((n_tok, TOP_K), jnp.int32),
        ),
    )(x2, wt)
    return scores, logits, ew, ei


# speedup vs baseline: 1.4637x; 1.0477x over previous
import jax
import jax.numpy as jnp
from jax.experimental import pallas as pl
from jax.experimental.pallas import tpu as pltpu

NUM_EXPERTS = 64
TOP_K = 8
BLOCK_T = 512


def _router_part(x, wt, lo, logits_ref, scores_ref, ew_ref, ei_ref):
    logits = jnp.dot(x, wt, preferred_element_type=jnp.float32)
    m = jnp.max(logits, axis=-1, keepdims=True)
    e = jnp.exp(logits - m)
    scores = e / jnp.sum(e, axis=-1, keepdims=True)
    logits_ref[0, lo:lo + BLOCK_T, :] = logits
    scores_ref[0, lo:lo + BLOCK_T, :] = scores

    s = scores
    colf = jax.lax.broadcasted_iota(jnp.int32, s.shape, 1).astype(jnp.float32)
    big = jnp.float32(NUM_EXPERTS)
    vals = []
    idxs = []
    for _ in range(TOP_K):
        mk = jnp.max(s, axis=-1, keepdims=True)
        ik = jnp.min(jnp.where(s == mk, colf, big), axis=-1, keepdims=True)
        vals.append(mk)
        idxs.append(ik)
        s = jnp.where(colf == ik, jnp.float32(-1.0), s)
    tv = jnp.concatenate(vals, axis=-1)
    ti = jnp.concatenate(idxs, axis=-1)
    ee = jnp.exp(tv - tv[:, :1])
    ew_ref[0, lo:lo + BLOCK_T, :] = ee / jnp.sum(ee, axis=-1, keepdims=True)
    ei_ref[0, lo:lo + BLOCK_T, :] = ti.astype(jnp.int32)


def _router_block(xa_ref, xb_ref, wt_ref,
                  logits_ref, scores_ref, ew_ref, ei_ref):
    wt = wt_ref[...]
    _router_part(xa_ref[...], wt, 0, logits_ref, scores_ref, ew_ref, ei_ref)
    _router_part(xb_ref[...], wt, BLOCK_T,
                 logits_ref, scores_ref, ew_ref, ei_ref)


def kernel(x, W):
    bs, sq, d = x.shape
    n_tok = bs * sq
    x2 = x.reshape(n_tok, d)
    wt = W.T
    n_steps = n_tok // (2 * BLOCK_T)
    E, K = NUM_EXPERTS, TOP_K
    T2 = 2 * BLOCK_T

    logits, scores, ew, ei = pl.pallas_call(
        _router_block,
        grid=(n_steps,),
        in_specs=[
            pl.BlockSpec((BLOCK_T, d), lambda i: (2 * i, 0)),
            pl.BlockSpec((BLOCK_T, d), lambda i: (2 * i + 1, 0)),
            pl.BlockSpec((d, E), lambda i: (0, 0)),
        ],
        out_specs=(
            pl.BlockSpec((1, T2, E), lambda i: (i, 0, 0)),
            pl.BlockSpec((1, T2, E), lambda i: (i, 0, 0)),
            pl.BlockSpec((1, T2, K), lambda i: (i, 0, 0)),
            pl.BlockSpec((1, T2, K), lambda i: (i, 0, 0)),
        ),
        out_shape=(
            jax.ShapeDtypeStruct((n_steps, T2, E), jnp.float32),
            jax.ShapeDtypeStruct((n_steps, T2, E), jnp.float32),
            jax.ShapeDtypeStruct((n_steps, T2, K), jnp.float32),
            jax.ShapeDtypeStruct((n_steps, T2, K), jnp.int32),
        ),
    )(x2, x2, wt)
    return (scores.reshape(n_tok, E), logits.reshape(n_tok, E),
            ew.reshape(n_tok, K), ei.reshape(n_tok, K))


# speedup vs baseline: 1.5243x; 1.0415x over previous
import jax
import jax.numpy as jnp
from jax.experimental import pallas as pl
from jax.experimental.pallas import tpu as pltpu

NUM_EXPERTS = 64
TOP_K = 8
BLOCK_T = 256
N_STREAMS = 4


def _router_part(x, wt, lo, logits_ref, scores_ref, ew_ref, ei_ref):
    logits = jnp.dot(x, wt, preferred_element_type=jnp.float32)
    m = jnp.max(logits, axis=-1, keepdims=True)
    e = jnp.exp(logits - m)
    scores = e / jnp.sum(e, axis=-1, keepdims=True)
    logits_ref[0, lo:lo + BLOCK_T, :] = logits
    scores_ref[0, lo:lo + BLOCK_T, :] = scores

    s = scores
    colf = jax.lax.broadcasted_iota(jnp.int32, s.shape, 1).astype(jnp.float32)
    big = jnp.float32(NUM_EXPERTS)
    vals = []
    idxs = []
    for _ in range(TOP_K):
        mk = jnp.max(s, axis=-1, keepdims=True)
        ik = jnp.min(jnp.where(s == mk, colf, big), axis=-1, keepdims=True)
        vals.append(mk)
        idxs.append(ik)
        s = jnp.where(colf == ik, jnp.float32(-1.0), s)
    tv = jnp.concatenate(vals, axis=-1)
    ti = jnp.concatenate(idxs, axis=-1)
    ee = jnp.exp(tv - tv[:, :1])
    ew_ref[0, lo:lo + BLOCK_T, :] = ee / jnp.sum(ee, axis=-1, keepdims=True)
    ei_ref[0, lo:lo + BLOCK_T, :] = ti.astype(jnp.int32)


def _router_block(*refs):
    x_refs = refs[:N_STREAMS]
    wt_ref = refs[N_STREAMS]
    logits_ref, scores_ref, ew_ref, ei_ref = refs[N_STREAMS + 1:]
    wt = wt_ref[...]
    for j, x_ref in enumerate(x_refs):
        _router_part(x_ref[...], wt, j * BLOCK_T,
                     logits_ref, scores_ref, ew_ref, ei_ref)


def kernel(x, W):
    bs, sq, d = x.shape
    n_tok = bs * sq
    x2 = x.reshape(n_tok, d)
    wt = W.T
    ns = N_STREAMS
    n_steps = n_tok // (ns * BLOCK_T)
    E, K = NUM_EXPERTS, TOP_K
    TT = ns * BLOCK_T

    def xmap(j):
        return lambda i: (ns * i + j, 0)

    logits, scores, ew, ei = pl.pallas_call(
        _router_block,
        grid=(n_steps,),
        in_specs=[pl.BlockSpec((BLOCK_T, d), xmap(j)) for j in range(ns)]
        + [pl.BlockSpec((d, E), lambda i: (0, 0))],
        out_specs=(
            pl.BlockSpec((1, TT, E), lambda i: (i, 0, 0)),
            pl.BlockSpec((1, TT, E), lambda i: (i, 0, 0)),
            pl.BlockSpec((1, TT, K), lambda i: (i, 0, 0)),
            pl.BlockSpec((1, TT, K), lambda i: (i, 0, 0)),
        ),
        out_shape=(
            jax.ShapeDtypeStruct((n_steps, TT, E), jnp.float32),
            jax.ShapeDtypeStruct((n_steps, TT, E), jnp.float32),
            jax.ShapeDtypeStruct((n_steps, TT, K), jnp.float32),
            jax.ShapeDtypeStruct((n_steps, TT, K), jnp.int32),
        ),
    )(*([x2] * ns), wt)
    return (scores.reshape(n_tok, E), logits.reshape(n_tok, E),
            ew.reshape(n_tok, K), ei.reshape(n_tok, K))


# speedup vs baseline: 1.5278x; 1.0023x over previous
import jax
import jax.numpy as jnp
from jax.experimental import pallas as pl
from jax.experimental.pallas import tpu as pltpu

NUM_EXPERTS = 64
TOP_K = 8
BLOCK_T = 256
N_STREAMS = 4


def _router_part(x, wt, lo, logits_ref, scores_ref, ew_ref, ei_ref):
    logits = jnp.dot(x, wt, preferred_element_type=jnp.float32)
    m = jnp.max(logits, axis=-1, keepdims=True)
    e = jnp.exp(logits - m)
    scores = e / jnp.sum(e, axis=-1, keepdims=True)
    logits_ref[0, lo:lo + BLOCK_T, :] = logits
    scores_ref[0, lo:lo + BLOCK_T, :] = scores

    s = scores
    colf = jax.lax.broadcasted_iota(jnp.int32, s.shape, 1).astype(jnp.float32)
    big = jnp.float32(NUM_EXPERTS)
    vals = []
    idxs = []
    for _ in range(TOP_K):
        mk = jnp.max(s, axis=-1, keepdims=True)
        ik = jnp.min(jnp.where(s == mk, colf, big), axis=-1, keepdims=True)
        vals.append(mk)
        idxs.append(ik)
        s = jnp.where(colf == ik, jnp.float32(-1.0), s)
    tv = jnp.concatenate(vals, axis=-1)
    ti = jnp.concatenate(idxs, axis=-1)
    ee = jnp.exp(tv - tv[:, :1])
    ew_ref[0, lo:lo + BLOCK_T, :] = ee / jnp.sum(ee, axis=-1, keepdims=True)
    ei_ref[0, lo:lo + BLOCK_T, :] = ti.astype(jnp.int32)


def _router_block(*refs):
    x_refs = refs[:N_STREAMS]
    wt_ref = refs[N_STREAMS]
    logits_ref, scores_ref, ew_ref, ei_ref = refs[N_STREAMS + 1:]
    wt = wt_ref[...]
    for j, x_ref in enumerate(x_refs):
        _router_part(x_ref[...], wt, j * BLOCK_T,
                     logits_ref, scores_ref, ew_ref, ei_ref)


def kernel(x, W):
    bs, sq, d = x.shape
    n_tok = bs * sq
    x2 = x.reshape(n_tok, d)
    wt = W.T
    ns = N_STREAMS
    n_steps = n_tok // (ns * BLOCK_T)
    E, K = NUM_EXPERTS, TOP_K
    TT = ns * BLOCK_T

    def xmap(j):
        return lambda i: (ns * i + j, 0)

    logits, scores, ew, ei = pl.pallas_call(
        _router_block,
        grid=(n_steps,),
        in_specs=[pl.BlockSpec((BLOCK_T, d), xmap(j)) for j in range(ns)]
        + [pl.BlockSpec((d, E), lambda i: (0, 0))],
        out_specs=(
            pl.BlockSpec((1, TT, E), lambda i: (i, 0, 0)),
            pl.BlockSpec((1, TT, E), lambda i: (i, 0, 0)),
            pl.BlockSpec((1, TT, K), lambda i: (i, 0, 0)),
            pl.BlockSpec((1, TT, K), lambda i: (i, 0, 0)),
        ),
        out_shape=(
            jax.ShapeDtypeStruct((n_steps, TT, E), jnp.float32),
            jax.ShapeDtypeStruct((n_steps, TT, E), jnp.float32),
            jax.ShapeDtypeStruct((n_steps, TT, K), jnp.float32),
            jax.ShapeDtypeStruct((n_steps, TT, K), jnp.int32),
        ),
        compiler_params=pltpu.CompilerParams(
            dimension_semantics=("parallel",)),
    )(*([x2] * ns), wt)
    return (scores.reshape(n_tok, E), logits.reshape(n_tok, E),
            ew.reshape(n_tok, K), ei.reshape(n_tok, K))
